# Initial kernel scaffold; baseline (speedup 1.0000x reference)
import functools

import jax
import jax.numpy as jnp
from jax import lax
from jax.experimental import pallas as pl
from jax.experimental.pallas import tpu as pltpu
from jax.experimental.pallas import tpu_sc as plsc

MP_ = 2
N_ = 10000
E_ = 320000
D_ = 128
H_ = 128

NC_ = 2
NS_ = 16
NW_ = NC_ * NS_
EPW_ = E_ // NW_
CHUNK_ = 80
NCHUNK_ = EPW_ // CHUNK_


def _sc_mesh():
    return plsc.VectorSubcoreMesh(core_axis_name="c", subcore_axis_name="s")


def _gather2_sc(tab0, tab1, idx0, idx1):
    @functools.partial(
        pl.kernel,
        out_type=(jax.ShapeDtypeStruct((E_, H_), jnp.float32),
                  jax.ShapeDtypeStruct((E_, H_), jnp.float32)),
        mesh=_sc_mesh(),
        scratch_types=[
            pltpu.VMEM((NCHUNK_, CHUNK_), jnp.int32),
            pltpu.VMEM((NCHUNK_, CHUNK_), jnp.int32),
            pltpu.VMEM((CHUNK_, H_), jnp.float32),
            pltpu.VMEM((CHUNK_, H_), jnp.float32),
            pltpu.SemaphoreType.DMA,
            pltpu.SemaphoreType.DMA,
        ],
    )
    def k(tab0_hbm, tab1_hbm, idx0_hbm, idx1_hbm, out0_hbm, out1_hbm,
          idx0_v, idx1_v, buf0, buf1, sem0, sem1):
        wid = lax.axis_index("s") * NC_ + lax.axis_index("c")
        pltpu.sync_copy(idx0_hbm.at[wid], idx0_v)
        pltpu.sync_copy(idx1_hbm.at[wid], idx1_v)
        base = wid * EPW_

        def body(j, carry):
            cp0 = pltpu.async_copy(tab0_hbm.at[idx0_v.at[j]], buf0, sem0)
            cp1 = pltpu.async_copy(tab1_hbm.at[idx1_v.at[j]], buf1, sem1)
            cp0.wait()
            cp1.wait()
            off = base + j * CHUNK_
            pltpu.sync_copy(buf0, out0_hbm.at[pl.ds(off, CHUNK_)])
            pltpu.sync_copy(buf1, out1_hbm.at[pl.ds(off, CHUNK_)])
            return carry

        lax.fori_loop(0, NCHUNK_, body, 0)

    return k(tab0, tab1, idx0, idx1)


def _scatter_sc(ea, idx1, zinit):
    @functools.partial(
        pl.kernel,
        out_type=jax.ShapeDtypeStruct((NC_, N_, D_), jnp.float32),
        mesh=_sc_mesh(),
        scratch_types=[
            pltpu.VMEM((NCHUNK_, CHUNK_), jnp.int32),
            pltpu.VMEM((CHUNK_, D_), jnp.float32),
            pltpu.VMEM_SHARED((N_, D_), jnp.float32),
        ],
    )
    def k(ea_hbm, idx_hbm, z_hbm, out_hbm, idx_v, buf, acc_sh):
        c = lax.axis_index("c")
        s = lax.axis_index("s")
        wid = s * NC_ + c
        rows_per_s = N_ // NS_
        pltpu.sync_copy(z_hbm.at[pl.ds(s * rows_per_s, rows_per_s)],
                        acc_sh.at[pl.ds(s * rows_per_s, rows_per_s)])
        pltpu.sync_copy(idx_hbm.at[wid], idx_v)
        plsc.subcore_barrier()
        base = wid * EPW_

        def body(j, carry):
            pltpu.sync_copy(ea_hbm.at[pl.ds(base + j * CHUNK_, CHUNK_)], buf)
            pltpu.sync_copy(buf, acc_sh.at[idx_v.at[j]], add=True)
            return carry

        lax.fori_loop(0, NCHUNK_, body, 0)
        plsc.subcore_barrier()
        pltpu.sync_copy(acc_sh.at[pl.ds(s * rows_per_s, rows_per_s)],
                        out_hbm.at[c].at[pl.ds(s * rows_per_s, rows_per_s)])

    return k(ea, idx1, zinit)


def _prep_tc(x, w1a, w1b):
    BN = 2000
    grid = (N_ // BN,)

    def body(x_ref, wa_ref, wb_ref, p_ref, q_ref):
        xb = x_ref[...]
        p_ref[...] = jnp.dot(xb, wa_ref[...], preferred_element_type=jnp.float32)
        q_ref[...] = jnp.dot(xb, wb_ref[...], preferred_element_type=jnp.float32)

    row = pl.BlockSpec((BN, D_), lambda i: (i, 0))
    w = pl.BlockSpec((D_, H_), lambda i: (0, 0))
    return pl.pallas_call(
        body, grid=grid,
        in_specs=[row, w, w],
        out_specs=[pl.BlockSpec((BN, H_), lambda i: (i, 0))] * 2,
        out_shape=[jax.ShapeDtypeStruct((N_, H_), jnp.float32)] * 2,
    )(x, w1a, w1b)


def _mlp_tail(h, w2, b2, w3, b3, g, bb):
    h = jnp.maximum(jnp.dot(h, w2, preferred_element_type=jnp.float32) + b2, 0.0)
    h = jnp.dot(h, w3, preferred_element_type=jnp.float32) + b3
    mu = jnp.mean(h, axis=-1, keepdims=True)
    var = jnp.mean((h - mu) ** 2, axis=-1, keepdims=True)
    return (h - mu) * lax.rsqrt(var + 1e-5) * g + bb


def _edge_mlp_tc(g1, g2, ea, w1c, b1, w2, b2, w3, b3, g, bb):
    BE = 1280
    grid = (E_ // BE,)

    def body(g1_ref, g2_ref, ea_ref, w1_ref, b1_ref, w2_ref, b2_ref,
             w3_ref, b3_ref, g_ref, bb_ref, out_ref):
        ea_b = ea_ref[...]
        h = (g1_ref[...] + g2_ref[...] + b1_ref[...]
             + jnp.dot(ea_b, w1_ref[...], preferred_element_type=jnp.float32))
        h = jnp.maximum(h, 0.0)
        out_ref[...] = _mlp_tail(h, w2_ref[...], b2_ref[...], w3_ref[...],
                                 b3_ref[...], g_ref[...], bb_ref[...]) + ea_b

    row = pl.BlockSpec((BE, H_), lambda i: (i, 0))
    w = pl.BlockSpec((H_, H_), lambda i: (0, 0))
    b = pl.BlockSpec((1, H_), lambda i: (0, 0))
    return pl.pallas_call(
        body, grid=grid,
        in_specs=[row, row, row, w, b, w, b, w, b, b, b],
        out_specs=pl.BlockSpec((BE, D_), lambda i: (i, 0)),
        out_shape=jax.ShapeDtypeStruct((E_, D_), jnp.float32),
    )(g1, g2, ea, w1c, b1.reshape(1, -1), w2, b2.reshape(1, -1),
      w3, b3.reshape(1, -1), g.reshape(1, -1), bb.reshape(1, -1))


def _node_mlp_tc(x, parts, w1a, w1b, b1, w2, b2, w3, b3, g, bb):
    BN = 2000
    grid = (N_ // BN,)

    def body(x_ref, p_ref, w1a_ref, w1b_ref, b1_ref, w2_ref, b2_ref,
             w3_ref, b3_ref, g_ref, bb_ref, out_ref):
        xb = x_ref[...]
        agg = p_ref[0] + p_ref[1]
        h = (jnp.dot(xb, w1a_ref[...], preferred_element_type=jnp.float32)
             + jnp.dot(agg, w1b_ref[...], preferred_element_type=jnp.float32)
             + b1_ref[...])
        h = jnp.maximum(h, 0.0)
        out_ref[...] = _mlp_tail(h, w2_ref[...], b2_ref[...], w3_ref[...],
                                 b3_ref[...], g_ref[...], bb_ref[...]) + xb

    row = pl.BlockSpec((BN, D_), lambda i: (i, 0))
    prow = pl.BlockSpec((NC_, BN, D_), lambda i: (0, i, 0))
    w = pl.BlockSpec((D_, H_), lambda i: (0, 0))
    b = pl.BlockSpec((1, H_), lambda i: (0, 0))
    return pl.pallas_call(
        body, grid=grid,
        in_specs=[row, prow, w, w, b, w, b, w, b, b, b],
        out_specs=row,
        out_shape=jax.ShapeDtypeStruct((N_, D_), jnp.float32),
    )(x, parts, w1a, w1b, b1.reshape(1, -1), w2, b2.reshape(1, -1),
      w3, b3.reshape(1, -1), g.reshape(1, -1), bb.reshape(1, -1))


def kernel(x, edge_indices, edge_attrs, eW1, eb1, eW2, eb2, eW3, eb3, eg, ebb,
           nW1, nb1, nW2, nb2, nW3, nb3, ng, nbb):
    ei = edge_indices[0].astype(jnp.int32)
    idx0 = ei[0].reshape(NW_, NCHUNK_, CHUNK_)
    idx1 = ei[1].reshape(NW_, NCHUNK_, CHUNK_)
    ea = edge_attrs[0]
    zinit = jnp.zeros((N_, D_), jnp.float32)

    for i in range(MP_):
        p_tab, q_tab = _prep_tc(x, eW1[i, :D_], eW1[i, D_:2 * D_])
        g1, g2 = _gather2_sc(p_tab, q_tab, idx0, idx1)
        ea = _edge_mlp_tc(g1, g2, ea, eW1[i, 2 * D_:], eb1[i], eW2[i], eb2[i],
                          eW3[i], eb3[i], eg[i], ebb[i])
        parts = _scatter_sc(ea, idx1, zinit)
        x = _node_mlp_tc(x, parts, nW1[i, :D_], nW1[i, D_:], nb1[i],
                         nW2[i], nb2[i], nW3[i], nb3[i], ng[i], nbb[i])
    return (x, ea[None])


# speedup vs baseline: 3.2860x; 3.2860x over previous
import functools

import jax
import jax.numpy as jnp
from jax import lax
from jax.experimental import pallas as pl
from jax.experimental.pallas import tpu as pltpu
from jax.experimental.pallas import tpu_sc as plsc

MP_ = 2
N_ = 10000
E_ = 320000
D_ = 128
H_ = 128

NC_ = 2
NS_ = 16
NW_ = NC_ * NS_
EPW_ = E_ // NW_
CHUNK_ = 80
NCHUNK_ = EPW_ // CHUNK_
NPAD_ = 10240


def _sc_mesh():
    return plsc.VectorSubcoreMesh(core_axis_name="c", subcore_axis_name="s")


def _gather2_sc(tab0, tab1, idx0, idx1):
    @functools.partial(
        pl.kernel,
        out_type=(jax.ShapeDtypeStruct((E_, H_), jnp.float32),
                  jax.ShapeDtypeStruct((E_, H_), jnp.float32)),
        mesh=_sc_mesh(),
        scratch_types=[
            pltpu.VMEM((NCHUNK_, CHUNK_), jnp.int32),
            pltpu.VMEM((NCHUNK_, CHUNK_), jnp.int32),
            pltpu.VMEM((CHUNK_, H_), jnp.float32),
            pltpu.VMEM((CHUNK_, H_), jnp.float32),
            pltpu.SemaphoreType.DMA,
            pltpu.SemaphoreType.DMA,
        ],
    )
    def k(tab0_hbm, tab1_hbm, idx0_hbm, idx1_hbm, out0_hbm, out1_hbm,
          idx0_v, idx1_v, buf0, buf1, sem0, sem1):
        wid = lax.axis_index("s") * NC_ + lax.axis_index("c")
        pltpu.sync_copy(idx0_hbm.at[wid], idx0_v)
        pltpu.sync_copy(idx1_hbm.at[wid], idx1_v)
        base = wid * EPW_

        def body(j, carry):
            cp0 = pltpu.async_copy(tab0_hbm.at[idx0_v.at[j]], buf0, sem0)
            cp1 = pltpu.async_copy(tab1_hbm.at[idx1_v.at[j]], buf1, sem1)
            cp0.wait()
            cp1.wait()
            off = base + j * CHUNK_
            pltpu.sync_copy(buf0, out0_hbm.at[pl.ds(off, CHUNK_)])
            pltpu.sync_copy(buf1, out1_hbm.at[pl.ds(off, CHUNK_)])
            return carry

        lax.fori_loop(0, NCHUNK_, body, 0)

    return k(tab0, tab1, idx0, idx1)


def _scatter_sc(ea, idx1, zinit):
    @functools.partial(
        pl.kernel,
        out_type=jax.ShapeDtypeStruct((NC_, NPAD_, D_), jnp.float32),
        mesh=_sc_mesh(),
        scratch_types=[
            pltpu.VMEM((NCHUNK_, CHUNK_), jnp.int32),
            pltpu.VMEM((CHUNK_, D_), jnp.float32),
            pltpu.VMEM_SHARED((NPAD_, D_), jnp.float32),
        ],
    )
    def k(ea_hbm, idx_hbm, z_hbm, out_hbm, idx_v, buf, acc_sh):
        c = lax.axis_index("c")
        s = lax.axis_index("s")
        wid = s * NC_ + c
        rows_per_s = NPAD_ // NS_
        pltpu.sync_copy(z_hbm.at[pl.ds(s * rows_per_s, rows_per_s)],
                        acc_sh.at[pl.ds(s * rows_per_s, rows_per_s)])
        pltpu.sync_copy(idx_hbm.at[wid], idx_v)
        plsc.subcore_barrier()
        base = wid * EPW_

        def body(j, carry):
            pltpu.sync_copy(ea_hbm.at[pl.ds(base + j * CHUNK_, CHUNK_)], buf)
            pltpu.sync_copy(buf, acc_sh.at[idx_v.at[j]], add=True)
            return carry

        lax.fori_loop(0, NCHUNK_, body, 0)
        plsc.subcore_barrier()
        pltpu.sync_copy(acc_sh.at[pl.ds(s * rows_per_s, rows_per_s)],
                        out_hbm.at[c].at[pl.ds(s * rows_per_s, rows_per_s)])

    return k(ea, idx1, zinit)


def _prep_tc(x, w1a, w1b):
    BN = 2000
    grid = (N_ // BN,)

    def body(x_ref, wa_ref, wb_ref, p_ref, q_ref):
        xb = x_ref[...]
        p_ref[...] = jnp.dot(xb, wa_ref[...], preferred_element_type=jnp.float32)
        q_ref[...] = jnp.dot(xb, wb_ref[...], preferred_element_type=jnp.float32)

    row = pl.BlockSpec((BN, D_), lambda i: (i, 0))
    w = pl.BlockSpec((D_, H_), lambda i: (0, 0))
    return pl.pallas_call(
        body, grid=grid,
        in_specs=[row, w, w],
        out_specs=[pl.BlockSpec((BN, H_), lambda i: (i, 0))] * 2,
        out_shape=[jax.ShapeDtypeStruct((N_, H_), jnp.float32)] * 2,
    )(x, w1a, w1b)


def _mlp_tail(h, w2, b2, w3, b3, g, bb):
    h = jnp.maximum(jnp.dot(h, w2, preferred_element_type=jnp.float32) + b2, 0.0)
    h = jnp.dot(h, w3, preferred_element_type=jnp.float32) + b3
    mu = jnp.mean(h, axis=-1, keepdims=True)
    var = jnp.mean((h - mu) ** 2, axis=-1, keepdims=True)
    return (h - mu) * lax.rsqrt(var + 1e-5) * g + bb


def _edge_mlp_tc(g1, g2, ea, w1c, b1, w2, b2, w3, b3, g, bb):
    BE = 1280
    grid = (E_ // BE,)

    def body(g1_ref, g2_ref, ea_ref, w1_ref, b1_ref, w2_ref, b2_ref,
             w3_ref, b3_ref, g_ref, bb_ref, out_ref):
        ea_b = ea_ref[...]
        h = (g1_ref[...] + g2_ref[...] + b1_ref[...]
             + jnp.dot(ea_b, w1_ref[...], preferred_element_type=jnp.float32))
        h = jnp.maximum(h, 0.0)
        out_ref[...] = _mlp_tail(h, w2_ref[...], b2_ref[...], w3_ref[...],
                                 b3_ref[...], g_ref[...], bb_ref[...]) + ea_b

    row = pl.BlockSpec((BE, H_), lambda i: (i, 0))
    w = pl.BlockSpec((H_, H_), lambda i: (0, 0))
    b = pl.BlockSpec((1, H_), lambda i: (0, 0))
    return pl.pallas_call(
        body, grid=grid,
        in_specs=[row, row, row, w, b, w, b, w, b, b, b],
        out_specs=pl.BlockSpec((BE, D_), lambda i: (i, 0)),
        out_shape=jax.ShapeDtypeStruct((E_, D_), jnp.float32),
    )(g1, g2, ea, w1c, b1.reshape(1, -1), w2, b2.reshape(1, -1),
      w3, b3.reshape(1, -1), g.reshape(1, -1), bb.reshape(1, -1))


def _node_mlp_tc(x, parts, w1a, w1b, b1, w2, b2, w3, b3, g, bb):
    BN = 2000
    grid = (N_ // BN,)

    def body(x_ref, p_ref, w1a_ref, w1b_ref, b1_ref, w2_ref, b2_ref,
             w3_ref, b3_ref, g_ref, bb_ref, out_ref):
        xb = x_ref[...]
        agg = p_ref[0] + p_ref[1]
        h = (jnp.dot(xb, w1a_ref[...], preferred_element_type=jnp.float32)
             + jnp.dot(agg, w1b_ref[...], preferred_element_type=jnp.float32)
             + b1_ref[...])
        h = jnp.maximum(h, 0.0)
        out_ref[...] = _mlp_tail(h, w2_ref[...], b2_ref[...], w3_ref[...],
                                 b3_ref[...], g_ref[...], bb_ref[...]) + xb

    row = pl.BlockSpec((BN, D_), lambda i: (i, 0))
    prow = pl.BlockSpec((NC_, BN, D_), lambda i: (0, i, 0))
    w = pl.BlockSpec((D_, H_), lambda i: (0, 0))
    b = pl.BlockSpec((1, H_), lambda i: (0, 0))
    return pl.pallas_call(
        body, grid=grid,
        in_specs=[row, prow, w, w, b, w, b, w, b, b, b],
        out_specs=row,
        out_shape=jax.ShapeDtypeStruct((N_, D_), jnp.float32),
    )(x, parts, w1a, w1b, b1.reshape(1, -1), w2, b2.reshape(1, -1),
      w3, b3.reshape(1, -1), g.reshape(1, -1), bb.reshape(1, -1))


def kernel(x, edge_indices, edge_attrs, eW1, eb1, eW2, eb2, eW3, eb3, eg, ebb,
           nW1, nb1, nW2, nb2, nW3, nb3, ng, nbb):
    ei = edge_indices[0].astype(jnp.int32)
    idx0 = ei[0].reshape(NW_, NCHUNK_, CHUNK_)
    idx1 = ei[1].reshape(NW_, NCHUNK_, CHUNK_)
    ea = edge_attrs[0]
    zinit = jnp.zeros((NPAD_, D_), jnp.float32)

    for i in range(MP_):
        p_tab, q_tab = _prep_tc(x, eW1[i, :D_], eW1[i, D_:2 * D_])
        g1, g2 = _gather2_sc(p_tab, q_tab, idx0, idx1)
        ea = _edge_mlp_tc(g1, g2, ea, eW1[i, 2 * D_:], eb1[i], eW2[i], eb2[i],
                          eW3[i], eb3[i], eg[i], ebb[i])
        parts = _scatter_sc(ea, idx1, zinit)
        x = _node_mlp_tc(x, parts, nW1[i, :D_], nW1[i, D_:], nb1[i],
                         nW2[i], nb2[i], nW3[i], nb3[i], ng[i], nbb[i])
    return (x, ea[None])
